# Initial kernel scaffold; baseline (speedup 1.0000x reference)
import jax
import jax.numpy as jnp
from jax.experimental import pallas as pl


def kernel(x, edge_index, W1, b1, W2, b2, W3, b3):
    raise NotImplementedError("write your pallas kernel here")


# speedup vs baseline: 7.9227x; 7.9227x over previous
import functools

import jax
import jax.numpy as jnp
from jax import lax
from jax.experimental import pallas as pl
from jax.experimental.pallas import tpu as pltpu
from jax.experimental.pallas import tpu_sc as plsc

F32 = jnp.float32

NC = 2
NS = 16
NW = NC * NS
K = 128

N = 10000
D = 128
NCLS = 40
TPAD = N + 8
NPART = 10240
ACCR = NPART
RPT = NPART // NS
RCH = 128

_mesh = plsc.VectorSubcoreMesh(core_axis_name="c", subcore_axis_name="s")


def _agg_body(nchunks, table, src, dst, zrows, out, sidx, didx, rows, zbuf,
              acc, gsem):
  c = lax.axis_index("c")
  s = lax.axis_index("s")
  wid = s * NC + c
  epw = nchunks * K
  base = wid * epw

  pltpu.sync_copy(zrows, zbuf)
  for j in range(RPT // RCH):
    pltpu.sync_copy(zbuf, acc.at[pl.ds(s * RPT + j * RCH, RCH)])
  plsc.subcore_barrier()

  def body(i, carry):
    e0 = base + i * K
    pltpu.sync_copy(src.at[pl.ds(e0, K)], sidx.at[0])
    pltpu.sync_copy(dst.at[pl.ds(e0, K)], didx.at[0])
    pltpu.async_copy(table.at[sidx.at[0]], rows, gsem).wait()
    pltpu.sync_copy(rows, acc.at[didx.at[0]], add=True)
    return carry

  lax.fori_loop(0, nchunks, body, 0)
  plsc.subcore_barrier()

  for j in range(RPT // RCH):
    r0 = s * RPT + j * RCH
    pltpu.sync_copy(acc.at[pl.ds(r0, RCH)], zbuf)
    pltpu.sync_copy(zbuf, out.at[c, pl.ds(r0, RCH)])


def _make_agg(nchunks):
  return functools.partial(
      pl.kernel,
      out_type=jax.ShapeDtypeStruct((NC, NPART, D), F32),
      mesh=_mesh,
      scratch_types=[
          pltpu.VMEM((1, K), jnp.int32),
          pltpu.VMEM((1, K), jnp.int32),
          pltpu.VMEM((K, D), F32),
          pltpu.VMEM((RCH, D), F32),
          pltpu.VMEM_SHARED((ACCR, D), F32),
          pltpu.SemaphoreType.DMA,
      ],
  )(functools.partial(_agg_body, nchunks))


def _deg_body(nchunks, dst, ones, zrows, out, didx, ones_v, zbuf, acc):
  c = lax.axis_index("c")
  s = lax.axis_index("s")
  wid = s * NC + c
  epw = nchunks * K
  base = wid * epw

  pltpu.sync_copy(ones, ones_v)
  pltpu.sync_copy(zrows, zbuf)
  for j in range(RPT // RCH):
    pltpu.sync_copy(zbuf, acc.at[pl.ds(s * RPT + j * RCH, RCH)])
  plsc.subcore_barrier()

  def body(i, carry):
    pltpu.sync_copy(dst.at[pl.ds(base + i * K, K)], didx.at[0])
    pltpu.sync_copy(ones_v, acc.at[didx.at[0]], add=True)
    return carry

  lax.fori_loop(0, nchunks, body, 0)
  plsc.subcore_barrier()

  for j in range(RPT // RCH):
    r0 = s * RPT + j * RCH
    pltpu.sync_copy(acc.at[pl.ds(r0, RCH)], zbuf)
    pltpu.sync_copy(zbuf, out.at[c, pl.ds(r0, RCH)])


def _make_deg(nchunks):
  return functools.partial(
      pl.kernel,
      out_type=jax.ShapeDtypeStruct((NC, NPART, D), F32),
      mesh=_mesh,
      scratch_types=[
          pltpu.VMEM((1, K), jnp.int32),
          pltpu.VMEM((K, D), F32),
          pltpu.VMEM((RCH, D), F32),
          pltpu.VMEM_SHARED((ACCR, D), F32),
      ],
  )(functools.partial(_deg_body, nchunks))


_RB = 1000
_GRID = N // _RB


def _k1_body(cnt0, cnt1, x, w, hs, dis16):
  deg = cnt0[:, 0:1] + cnt1[:, 0:1] + 1.0
  dis = lax.rsqrt(deg)
  hs[:] = jnp.dot(x[:], w[:], preferred_element_type=F32) * dis
  dis16[:] = jnp.broadcast_to(dis, (_RB, 16))


def _k1_call(cnt0, cnt1, x, w):
  return pl.pallas_call(
      _k1_body,
      grid=(_GRID,),
      in_specs=[
          pl.BlockSpec((_RB, D), lambda i: (i, 0)),
          pl.BlockSpec((_RB, D), lambda i: (i, 0)),
          pl.BlockSpec((_RB, D), lambda i: (i, 0)),
          pl.BlockSpec((D, D), lambda i: (0, 0)),
      ],
      out_specs=[
          pl.BlockSpec((_RB, D), lambda i: (i, 0)),
          pl.BlockSpec((_RB, 16), lambda i: (i, 0)),
      ],
      out_shape=[
          jax.ShapeDtypeStruct((N, D), F32),
          jax.ShapeDtypeStruct((N, 16), F32),
      ],
  )(cnt0, cnt1, x, w)


def _k2_body(dis16, acc0, acc1, hsp, b, w, o):
  dis = dis16[:, 0:1]
  u = jnp.maximum(dis * (acc0[:] + acc1[:] + hsp[:]) + b[:], 0.0)
  o[:] = jnp.dot(u, w[:], preferred_element_type=F32) * dis


def _k2_call(dis16, acc0, acc1, hsp, b, w):
  return pl.pallas_call(
      _k2_body,
      grid=(_GRID,),
      in_specs=[
          pl.BlockSpec((_RB, 16), lambda i: (i, 0)),
          pl.BlockSpec((_RB, D), lambda i: (i, 0)),
          pl.BlockSpec((_RB, D), lambda i: (i, 0)),
          pl.BlockSpec((_RB, D), lambda i: (i, 0)),
          pl.BlockSpec((1, D), lambda i: (0, 0)),
          pl.BlockSpec((D, D), lambda i: (0, 0)),
      ],
      out_specs=pl.BlockSpec((_RB, D), lambda i: (i, 0)),
      out_shape=jax.ShapeDtypeStruct((N, D), F32),
  )(dis16, acc0, acc1, hsp, b, w)


def _k3_body(dis16, acc0, acc1, hs3, b, o):
  dis = dis16[:, 0:1]
  t = dis * (acc0[:] + acc1[:] + hs3[:]) + b[:]
  col = lax.broadcasted_iota(jnp.int32, t.shape, 1)
  valid = col < NCLS
  tm = jnp.where(valid, t, -jnp.inf)
  m = jnp.max(tm, axis=1, keepdims=True)
  e = jnp.where(valid, jnp.exp(t - m), 0.0)
  lse = jnp.log(jnp.sum(e, axis=1, keepdims=True))
  o[:] = (t - m - lse)[:, :NCLS]


def _k3_call(dis16, acc0, acc1, hs3, b):
  return pl.pallas_call(
      _k3_body,
      grid=(_GRID,),
      in_specs=[
          pl.BlockSpec((_RB, 16), lambda i: (i, 0)),
          pl.BlockSpec((_RB, D), lambda i: (i, 0)),
          pl.BlockSpec((_RB, D), lambda i: (i, 0)),
          pl.BlockSpec((_RB, D), lambda i: (i, 0)),
          pl.BlockSpec((1, D), lambda i: (0, 0)),
      ],
      out_specs=pl.BlockSpec((_RB, NCLS), lambda i: (i, 0)),
      out_shape=jax.ShapeDtypeStruct((N, NCLS), F32),
  )(dis16, acc0, acc1, hs3, b)


def _pad_table(hs):
  return jnp.concatenate([hs, jnp.zeros((TPAD - N, D), F32)], axis=0)


def kernel(x, edge_index, W1, b1, W2, b2, W3, b3):
  E = edge_index.shape[1]
  nchunks = -(-E // (NW * K))
  e_pad = nchunks * NW * K

  src = jnp.concatenate(
      [edge_index[0], jnp.full((e_pad - E,), N, jnp.int32)])
  dst = jnp.concatenate(
      [edge_index[1], jnp.full((e_pad - E,), N, jnp.int32)])

  zrows_d = jnp.zeros((RCH, D), F32)
  ones_d = jnp.ones((K, D), F32)

  agg = _make_agg(nchunks)
  deg = _make_deg(nchunks)

  cnt = deg(dst, ones_d, zrows_d)
  hs1, dis16 = _k1_call(cnt[0], cnt[1], x, W1)
  acc1 = agg(_pad_table(hs1), src, dst, zrows_d)
  hs2 = _k2_call(dis16, acc1[0], acc1[1], hs1, b1.reshape(1, D), W2)
  acc2 = agg(_pad_table(hs2), src, dst, zrows_d)

  W3p = jnp.zeros((D, D), F32).at[:, :NCLS].set(W3)
  b3p = jnp.zeros((1, D), F32).at[0, :NCLS].set(b3)
  hs3 = _k2_call(dis16, acc2[0], acc2[1], hs2, b2.reshape(1, D), W3p)
  acc3 = agg(_pad_table(hs3), src, dst, zrows_d)

  return _k3_call(dis16, acc3[0], acc3[1], hs3, b3p)
